# baseline (speedup 1.0000x reference)
import functools

import jax
import jax.numpy as jnp
from jax import lax
from jax.experimental import pallas as pl
from jax.experimental.pallas import tpu as pltpu
from jax.experimental.pallas import tpu_sc as plsc

BATCH = 16384
FIELDS = 26
EMB_DIM = 16
OUT_DIM = 16

NC = 2
NS = 16
NW = NC * NS

TOTAL = BATCH * FIELDS
B_PER_W = TOTAL // NW
NBUF = 6
CHUNK = 1024
N_CHUNKS = B_PER_W // CHUNK


def _sc_gather(table, idx_flat):
    mesh = plsc.VectorSubcoreMesh(core_axis_name="c", subcore_axis_name="s")

    scratch = (
        [pltpu.VMEM((CHUNK,), jnp.int32) for _ in range(NBUF)]
        + [pltpu.VMEM((CHUNK, EMB_DIM), jnp.float32) for _ in range(NBUF)]
        + [pltpu.SemaphoreType.DMA for _ in range(2 * NBUF)]
    )

    @functools.partial(
        pl.kernel,
        mesh=mesh,
        out_type=jax.ShapeDtypeStruct((TOTAL, EMB_DIM), jnp.float32),
        compiler_params=pltpu.CompilerParams(use_tc_tiling_on_sc=False),
        scratch_types=scratch,
    )
    def gather_kernel(table_hbm, idx_hbm, out_hbm, *bufs):
        idx_v = bufs[:NBUF]
        rows_v = bufs[NBUF:2 * NBUF]
        gsem = bufs[2 * NBUF:3 * NBUF]
        wsem = bufs[3 * NBUF:]
        wid = lax.axis_index("s") * NC + lax.axis_index("c")
        base_w = wid * B_PER_W

        def idx_load(c, b):
            pltpu.sync_copy(idx_hbm.at[pl.ds(base_w + c * CHUNK, CHUNK)],
                            idx_v[b])

        def gather(b):
            return pltpu.make_async_copy(table_hbm.at[idx_v[b]], rows_v[b],
                                         gsem[b])

        def write(c, b):
            return pltpu.make_async_copy(
                rows_v[b], out_hbm.at[pl.ds(base_w + c * CHUNK, CHUNK)],
                wsem[b])

        for c in range(min(NBUF, N_CHUNKS)):
            idx_load(c, c)
            gather(c).start()
        for c in range(N_CHUNKS):
            b = c % NBUF
            gather(b).wait()
            write(c, b).start()
            n = c + NBUF
            if n < N_CHUNKS:
                write(c, b).wait()
                idx_load(n, b)
                gather(b).start()
        for c in range(max(N_CHUNKS - NBUF, 0), N_CHUNKS):
            write(c, c % NBUF).wait()

    return gather_kernel(table, idx_flat)


K_DIM = FIELDS * EMB_DIM
MM_BLK = 2048

BAND = 13
PACK_ROWS = TOTAL * EMB_DIM // 128
PACK_BLK = MM_BLK * K_DIM // 128
R_SUB = PACK_BLK // BAND

import numpy as _np

_POS = _np.arange(BAND * 128)
_K_OF_POS = _POS % K_DIM
_B_OF_POS = _POS // K_DIM
_COLS = (16 * _B_OF_POS[:, None] + _np.arange(OUT_DIM)[None, :])


def _build_wstack(W):
    wg = W[_K_OF_POS]
    ws = jnp.zeros((BAND * 128, 4 * OUT_DIM), jnp.float32)
    ws = ws.at[_np.arange(BAND * 128)[:, None], _COLS].set(wg)
    return ws.reshape(BAND, 128, 4 * OUT_DIM)


def _mm_kernel(x_ref, ws_ref, b_ref, o_ref):
    acc = jnp.zeros((R_SUB, 4 * OUT_DIM), jnp.float32)
    for j in range(BAND):
        xj = x_ref[pl.ds(j, R_SUB, BAND), :]
        acc += jnp.dot(xj, ws_ref[j],
                       preferred_element_type=jnp.float32)
    acc += b_ref[...]
    for q in range(4):
        o_ref[pl.ds(q, R_SUB, 4), :] = acc[:, q * OUT_DIM:(q + 1) * OUT_DIM]


def _tc_project(packed, wstack, btile):
    grid = (BATCH // MM_BLK,)
    return pl.pallas_call(
        _mm_kernel,
        grid=grid,
        in_specs=[
            pl.BlockSpec((PACK_BLK, 128), lambda i: (i, 0)),
            pl.BlockSpec((BAND, 128, 4 * OUT_DIM), lambda i: (0, 0, 0)),
            pl.BlockSpec((1, 4 * OUT_DIM), lambda i: (0, 0)),
        ],
        out_specs=pl.BlockSpec((MM_BLK, OUT_DIM), lambda i: (i, 0)),
        out_shape=jax.ShapeDtypeStruct((BATCH, OUT_DIM), jnp.float32),
    )(packed, wstack, btile)


def kernel(inputs, table, W, b):
    idx_flat = inputs.reshape(-1).astype(jnp.int32)
    gathered = _sc_gather(table, idx_flat)
    packed = gathered.reshape(PACK_ROWS, 128)
    wstack = _build_wstack(W)
    btile = jnp.tile(b, 4).reshape(1, 4 * OUT_DIM)
    return _tc_project(packed, wstack, btile)


# speedup vs baseline: 1.1577x; 1.1577x over previous
import functools

import jax
import jax.numpy as jnp
from jax import lax
from jax.experimental import pallas as pl
from jax.experimental.pallas import tpu as pltpu
from jax.experimental.pallas import tpu_sc as plsc

BATCH = 16384
FIELDS = 26
EMB_DIM = 16
OUT_DIM = 16

NC = 2
NS = 16
NW = NC * NS

TOTAL = BATCH * FIELDS
B_PER_W = TOTAL // NW
NBUF = 6
CHUNK = 1024
N_CHUNKS = B_PER_W // CHUNK


def _sc_gather(table, idx_flat):
    mesh = plsc.VectorSubcoreMesh(core_axis_name="c", subcore_axis_name="s")

    scratch = (
        [pltpu.VMEM((CHUNK,), jnp.int32) for _ in range(NBUF)]
        + [pltpu.VMEM((CHUNK, EMB_DIM), jnp.float32) for _ in range(NBUF)]
        + [pltpu.SemaphoreType.DMA for _ in range(2 * NBUF)]
    )

    @functools.partial(
        pl.kernel,
        mesh=mesh,
        out_type=jax.ShapeDtypeStruct((TOTAL, EMB_DIM), jnp.float32),
        compiler_params=pltpu.CompilerParams(use_tc_tiling_on_sc=False),
        scratch_types=scratch,
    )
    def gather_kernel(table_hbm, idx_hbm, out_hbm, *bufs):
        idx_v = bufs[:NBUF]
        rows_v = bufs[NBUF:2 * NBUF]
        gsem = bufs[2 * NBUF:3 * NBUF]
        wsem = bufs[3 * NBUF:]
        wid = lax.axis_index("s") * NC + lax.axis_index("c")
        base_w = wid * B_PER_W

        def idx_load(c, b):
            pltpu.sync_copy(idx_hbm.at[pl.ds(base_w + c * CHUNK, CHUNK)],
                            idx_v[b])

        def gather(b):
            return pltpu.make_async_copy(table_hbm.at[idx_v[b]], rows_v[b],
                                         gsem[b])

        def write(c, b):
            return pltpu.make_async_copy(
                rows_v[b], out_hbm.at[pl.ds(base_w + c * CHUNK, CHUNK)],
                wsem[b])

        for c in range(min(NBUF, N_CHUNKS)):
            idx_load(c, c)
            gather(c).start()
        for c in range(N_CHUNKS):
            b = c % NBUF
            gather(b).wait()
            write(c, b).start()
            n = c + NBUF
            if n < N_CHUNKS:
                write(c, b).wait()
                idx_load(n, b)
                gather(b).start()
        for c in range(max(N_CHUNKS - NBUF, 0), N_CHUNKS):
            write(c, c % NBUF).wait()

    return gather_kernel(table, idx_flat)


K_DIM = FIELDS * EMB_DIM
MM_BLK = 2048

BAND = 13
PACK_ROWS = TOTAL * EMB_DIM // 128
PACK_BLK = MM_BLK * K_DIM // 128
R_SUB = PACK_BLK // BAND

import numpy as _np

_POS = _np.arange(BAND * 128)
_K_OF_POS = _POS % K_DIM
_B_OF_POS = _POS // K_DIM
_MASK = (_B_OF_POS[:, None] ==
         _np.repeat(_np.arange(4), OUT_DIM)[None, :]).astype(_np.float32)


def _build_wstack(W):
    wg = jnp.tile(W[_K_OF_POS], (1, 4))
    ws = wg * jnp.asarray(_MASK)
    return ws.reshape(BAND, 128, 4 * OUT_DIM)


def _mm_kernel(x_ref, ws_ref, b_ref, o_ref):
    acc = jnp.zeros((R_SUB, 4 * OUT_DIM), jnp.float32)
    for j in range(BAND):
        xj = x_ref[pl.ds(j, R_SUB, BAND), :]
        acc += jnp.dot(xj, ws_ref[j],
                       preferred_element_type=jnp.float32)
    acc += b_ref[...]
    for q in range(4):
        o_ref[pl.ds(q, R_SUB, 4), :] = acc[:, q * OUT_DIM:(q + 1) * OUT_DIM]


def _tc_project(packed, wstack, btile):
    grid = (BATCH // MM_BLK,)
    return pl.pallas_call(
        _mm_kernel,
        grid=grid,
        in_specs=[
            pl.BlockSpec((PACK_BLK, 128), lambda i: (i, 0)),
            pl.BlockSpec((BAND, 128, 4 * OUT_DIM), lambda i: (0, 0, 0)),
            pl.BlockSpec((1, 4 * OUT_DIM), lambda i: (0, 0)),
        ],
        out_specs=pl.BlockSpec((MM_BLK, OUT_DIM), lambda i: (i, 0)),
        out_shape=jax.ShapeDtypeStruct((BATCH, OUT_DIM), jnp.float32),
    )(packed, wstack, btile)


def kernel(inputs, table, W, b):
    idx_flat = inputs.reshape(-1).astype(jnp.int32)
    gathered = _sc_gather(table, idx_flat)
    packed = gathered.reshape(PACK_ROWS, 128)
    wstack = _build_wstack(W)
    btile = jnp.tile(b, 4).reshape(1, 4 * OUT_DIM)
    return _tc_project(packed, wstack, btile)
